# baseline (speedup 1.0000x reference)
import jax
import jax.numpy as jnp
from jax import lax
from jax.experimental import pallas as pl
from jax.experimental.pallas import tpu as pltpu

_N = 1000
_K1 = 7 * 7 * 256
_H = 1024
_NC = 81
_KBLK = 1792
_NKB = _K1 // _KBLK
_BN_EPS = 1e-3


def _bn_relu(h, gamma, beta):
    mean = jnp.mean(h, axis=0, keepdims=True)
    var = jnp.mean((h - mean) * (h - mean), axis=0, keepdims=True)
    inv = lax.rsqrt(var + _BN_EPS)
    return jnp.maximum((h - mean) * inv * gamma + beta, 0.0)


def _body(x_ref, w1_ref, b1_ref, g1_ref, be1_ref, w2_ref, b2_ref, g2_ref,
          be2_ref, lw_ref, lb_ref, dw_ref, db_ref,
          logits_ref, probs_ref, deltas_ref, acc_ref):
    k = pl.program_id(0)
    part = jnp.dot(x_ref[...].astype(jnp.bfloat16),
                   w1_ref[...].astype(jnp.bfloat16),
                   preferred_element_type=jnp.float32)

    @pl.when(k == 0)
    def _():
        acc_ref[...] = part

    @pl.when(k != 0)
    def _():
        acc_ref[...] += part

    @pl.when(k == _NKB - 1)
    def _():
        h1 = acc_ref[...] + b1_ref[...]
        x1 = _bn_relu(h1, g1_ref[...], be1_ref[...])
        h2 = jnp.dot(x1.astype(jnp.bfloat16),
                     w2_ref[...].astype(jnp.bfloat16),
                     preferred_element_type=jnp.float32)
        h2 = h2 + b2_ref[...]
        x2 = _bn_relu(h2, g2_ref[...], be2_ref[...])
        logits = jnp.dot(x2, lw_ref[...], preferred_element_type=jnp.float32)
        logits = logits + lb_ref[...]
        logits_ref[...] = logits
        m = jnp.max(logits, axis=-1, keepdims=True)
        e = jnp.exp(logits - m)
        probs_ref[...] = e / jnp.sum(e, axis=-1, keepdims=True)
        d = jnp.dot(x2, dw_ref[...], preferred_element_type=jnp.float32)
        deltas_ref[...] = d + db_ref[...]


def kernel(pooled_rois, conv1_w, conv1_b, bn1_gamma, bn1_beta, conv2_w,
           conv2_b, bn2_gamma, bn2_beta, logits_w, logits_b, delta_w,
           delta_b):
    n = pooled_rois.shape[0]
    x = pooled_rois.reshape(n, _K1)
    w1 = conv1_w.reshape(_K1, _H)
    w2 = conv2_w.reshape(_H, _H)
    row = lambda v: v.reshape(1, -1)

    full = lambda shape: pl.BlockSpec(shape, lambda i: (0, 0))
    logits, probs, deltas = pl.pallas_call(
        _body,
        grid=(_NKB,),
        in_specs=[
            pl.BlockSpec((n, _KBLK), lambda i: (0, i)),
            pl.BlockSpec((_KBLK, _H), lambda i: (i, 0)),
            full((1, _H)), full((1, _H)), full((1, _H)),
            full((_H, _H)),
            full((1, _H)), full((1, _H)), full((1, _H)),
            full((_H, _NC)), full((1, _NC)),
            full((_H, 4 * _NC)), full((1, 4 * _NC)),
        ],
        out_specs=[
            full((n, _NC)),
            full((n, _NC)),
            full((n, 4 * _NC)),
        ],
        out_shape=[
            jax.ShapeDtypeStruct((n, _NC), jnp.float32),
            jax.ShapeDtypeStruct((n, _NC), jnp.float32),
            jax.ShapeDtypeStruct((n, 4 * _NC), jnp.float32),
        ],
        scratch_shapes=[pltpu.VMEM((n, _H), jnp.float32)],
        compiler_params=pltpu.CompilerParams(
            dimension_semantics=("arbitrary",),
            vmem_limit_bytes=120 * 1024 * 1024,
        ),
    )(x, w1, row(conv1_b), row(bn1_gamma), row(bn1_beta), w2, row(conv2_b),
      row(bn2_gamma), row(bn2_beta), logits_w, row(logits_b), delta_w,
      row(delta_b))
    return logits, probs, deltas.reshape(n, _NC, 4)


# speedup vs baseline: 1.2582x; 1.2582x over previous
import jax
import jax.numpy as jnp
from jax import lax
from jax.experimental import pallas as pl
from jax.experimental.pallas import tpu as pltpu

_N = 1000
_H = 1024
_NC = 81
_TAPS = 49
_BN_EPS = 1e-3


def _bn_relu(h, gamma, beta):
    mean = jnp.mean(h, axis=0, keepdims=True)
    var = jnp.mean((h - mean) * (h - mean), axis=0, keepdims=True)
    inv = lax.rsqrt(var + _BN_EPS)
    return jnp.maximum((h - mean) * inv * gamma + beta, 0.0)


def _body(x_hbm, w1_ref, b1_ref, g1_ref, be1_ref, w2_ref, b2_ref, g2_ref,
          be2_ref, lw_ref, lb_ref, dw_ref, db_ref,
          logits_ref, probs_ref, deltas_ref, acc_ref, xbuf, sems):
    step = pl.program_id(0)
    slot = lax.rem(step, 2)

    @pl.when(step == 0)
    def _():
        pltpu.make_async_copy(x_hbm.at[:, 0, 0, :], xbuf.at[0],
                              sems.at[0]).start()

    @pl.when(step < _TAPS - 1)
    def _():
        nxt = step + 1
        pltpu.make_async_copy(
            x_hbm.at[:, lax.div(nxt, 7), lax.rem(nxt, 7), :],
            xbuf.at[lax.rem(nxt, 2)], sems.at[lax.rem(nxt, 2)]).start()

    pltpu.make_async_copy(
        x_hbm.at[:, lax.div(step, 7), lax.rem(step, 7), :],
        xbuf.at[slot], sems.at[slot]).wait()

    part = jnp.dot(xbuf[slot].astype(jnp.bfloat16),
                   w1_ref[0, 0].astype(jnp.bfloat16),
                   preferred_element_type=jnp.float32)

    @pl.when(step == 0)
    def _():
        acc_ref[...] = part

    @pl.when(step != 0)
    def _():
        acc_ref[...] += part

    @pl.when(step == _TAPS - 1)
    def _():
        h1 = acc_ref[...] + b1_ref[...]
        x1 = _bn_relu(h1, g1_ref[...], be1_ref[...])
        h2 = jnp.dot(x1.astype(jnp.bfloat16),
                     w2_ref[0, 0].astype(jnp.bfloat16),
                     preferred_element_type=jnp.float32)
        h2 = h2 + b2_ref[...]
        x2 = _bn_relu(h2, g2_ref[...], be2_ref[...])
        logits = jnp.dot(x2, lw_ref[...], preferred_element_type=jnp.float32)
        logits = logits + lb_ref[...]
        logits_ref[...] = logits
        m = jnp.max(logits, axis=-1, keepdims=True)
        e = jnp.exp(logits - m)
        probs_ref[...] = e / jnp.sum(e, axis=-1, keepdims=True)
        d = jnp.dot(x2, dw_ref[...], preferred_element_type=jnp.float32)
        deltas_ref[...] = d + db_ref[...]


def kernel(pooled_rois, conv1_w, conv1_b, bn1_gamma, bn1_beta, conv2_w,
           conv2_b, bn2_gamma, bn2_beta, logits_w, logits_b, delta_w,
           delta_b):
    n = pooled_rois.shape[0]
    row = lambda v: v.reshape(1, -1)

    full = lambda shape: pl.BlockSpec(shape, lambda s: (0,) * len(shape))
    logits, probs, deltas = pl.pallas_call(
        _body,
        grid=(_TAPS,),
        in_specs=[
            pl.BlockSpec(memory_space=pl.ANY),
            pl.BlockSpec((1, 1, 256, _H),
                         lambda s: (s // 7, s % 7, 0, 0)),
            full((1, _H)), full((1, _H)), full((1, _H)),
            pl.BlockSpec((1, 1, _H, _H), lambda s: (0, 0, 0, 0)),
            full((1, _H)), full((1, _H)), full((1, _H)),
            full((_H, _NC)), full((1, _NC)),
            full((_H, 4 * _NC)), full((1, 4 * _NC)),
        ],
        out_specs=[
            full((n, _NC)),
            full((n, _NC)),
            full((n, 4 * _NC)),
        ],
        out_shape=[
            jax.ShapeDtypeStruct((n, _NC), jnp.float32),
            jax.ShapeDtypeStruct((n, _NC), jnp.float32),
            jax.ShapeDtypeStruct((n, 4 * _NC), jnp.float32),
        ],
        scratch_shapes=[
            pltpu.VMEM((n, _H), jnp.float32),
            pltpu.VMEM((2, n, 256), jnp.float32),
            pltpu.SemaphoreType.DMA((2,)),
        ],
        compiler_params=pltpu.CompilerParams(
            dimension_semantics=("arbitrary",),
            vmem_limit_bytes=100 * 1024 * 1024,
        ),
    )(pooled_rois, conv1_w, row(conv1_b), row(bn1_gamma), row(bn1_beta),
      conv2_w, row(conv2_b), row(bn2_gamma), row(bn2_beta), logits_w,
      row(logits_b), delta_w, row(delta_b))
    return logits, probs, deltas.reshape(n, _NC, 4)
